# baseline (speedup 1.0000x reference)
import functools

import jax
import jax.numpy as jnp
from jax.experimental import pallas as pl
from jax.experimental.pallas import tpu as pltpu

_DIM = 2048
_NE = 8
_TOPK = 2
_BLK = 1024


def _route(st):
    m = jnp.max(st, axis=0, keepdims=True)
    e = jnp.exp(st - m)
    p = e * (1.0 / jnp.sum(e, axis=0, keepdims=True))

    iota = jax.lax.broadcasted_iota(jnp.int32, st.shape, 0)
    ninf = jnp.float32(-jnp.inf)
    big = jnp.int32(_NE)

    p1 = jnp.max(p, axis=0, keepdims=True)
    i1 = jnp.min(jnp.where(p == p1, iota, big), axis=0, keepdims=True)
    pm = jnp.where(iota == i1, ninf, p)
    p2 = jnp.max(pm, axis=0, keepdims=True)
    i2 = jnp.min(jnp.where(pm == p2, iota, big), axis=0, keepdims=True)

    w1 = jnp.max(jnp.where(iota == i1, st, ninf), axis=0, keepdims=True)
    w2 = jnp.max(jnp.where(iota == i2, st, ninf), axis=0, keepdims=True)
    return (jnp.concatenate([i1, i2], axis=0),
            jnp.concatenate([w1, w2], axis=0))


def _router_body(xa_ref, xb_ref, w_ref, ia_ref, wa_ref, ib_ref, wb_ref):
    w = w_ref[...]
    dn = (((1,), (1,)), ((), ()))
    sa = jax.lax.dot_general(w, xa_ref[...], dn,
                             preferred_element_type=jnp.float32)
    ia_ref[...], wa_ref[...] = _route(sa)
    sb = jax.lax.dot_general(w, xb_ref[...], dn,
                             preferred_element_type=jnp.float32)
    ib_ref[...], wb_ref[...] = _route(sb)


@jax.jit
def kernel(x, W):
    T = x.shape[0]
    half = T // 2
    nblk = half // _BLK
    xa = x[:half]
    xb = x[half:]
    ia, wa, ib, wb = pl.pallas_call(
        _router_body,
        grid=(nblk,),
        in_specs=[
            pl.BlockSpec((_BLK, _DIM), lambda i: (i, 0)),
            pl.BlockSpec((_BLK, _DIM), lambda i: (i, 0)),
            pl.BlockSpec((_NE, _DIM), lambda i: (0, 0)),
        ],
        out_specs=[
            pl.BlockSpec((_TOPK, _BLK), lambda i: (0, i)),
            pl.BlockSpec((_TOPK, _BLK), lambda i: (0, i)),
            pl.BlockSpec((_TOPK, _BLK), lambda i: (0, i)),
            pl.BlockSpec((_TOPK, _BLK), lambda i: (0, i)),
        ],
        out_shape=[
            jax.ShapeDtypeStruct((_TOPK, half), jnp.int32),
            jax.ShapeDtypeStruct((_TOPK, half), jnp.float32),
            jax.ShapeDtypeStruct((_TOPK, half), jnp.int32),
            jax.ShapeDtypeStruct((_TOPK, half), jnp.float32),
        ],
    )(xa, xb, W)
    idx_t = jnp.concatenate([ia, ib], axis=1)
    wgt_t = jnp.concatenate([wa, wb], axis=1)
    return idx_t.T, wgt_t.T


# speedup vs baseline: 2.9491x; 2.9491x over previous
import functools

import jax
import jax.numpy as jnp
from jax.experimental import pallas as pl
from jax.experimental.pallas import tpu as pltpu

_DIM = 2048
_NE = 8
_TOPK = 2
_BLK = 1024


def _route(st):
    m = jnp.max(st, axis=0, keepdims=True)
    e = jnp.exp(st - m)
    p = e * (1.0 / jnp.sum(e, axis=0, keepdims=True))

    iota = jax.lax.broadcasted_iota(jnp.int32, st.shape, 0)
    ninf = jnp.float32(-jnp.inf)
    big = jnp.int32(_NE)

    p1 = jnp.max(p, axis=0, keepdims=True)
    i1 = jnp.min(jnp.where(p == p1, iota, big), axis=0, keepdims=True)
    pm = jnp.where(iota == i1, ninf, p)
    p2 = jnp.max(pm, axis=0, keepdims=True)
    i2 = jnp.min(jnp.where(pm == p2, iota, big), axis=0, keepdims=True)

    w1 = jnp.max(jnp.where(iota == i1, st, ninf), axis=0, keepdims=True)
    w2 = jnp.max(jnp.where(iota == i2, st, ninf), axis=0, keepdims=True)
    return (jnp.concatenate([i1, i2], axis=0),
            jnp.concatenate([w1, w2], axis=0))


def _router_body(xa_ref, xb_ref, w_ref, ia_ref, wa_ref, ib_ref, wb_ref):
    w = w_ref[...]
    dn = (((1,), (1,)), ((), ()))
    sa = jax.lax.dot_general(w, xa_ref[...], dn,
                             preferred_element_type=jnp.float32)
    ia_ref[...], wa_ref[...] = _route(sa)
    sb = jax.lax.dot_general(w, xb_ref[...], dn,
                             preferred_element_type=jnp.float32)
    ib_ref[...], wb_ref[...] = _route(sb)


@jax.jit
def kernel(x, W):
    T = x.shape[0]
    half = T // 2
    nblk = half // _BLK
    ia, wa, ib, wb = pl.pallas_call(
        _router_body,
        grid=(nblk,),
        in_specs=[
            pl.BlockSpec((_BLK, _DIM), lambda i: (i, 0)),
            pl.BlockSpec((_BLK, _DIM), lambda i: (i + nblk, 0)),
            pl.BlockSpec((_NE, _DIM), lambda i: (0, 0)),
        ],
        out_specs=[
            pl.BlockSpec((_TOPK, _BLK), lambda i: (0, i)),
            pl.BlockSpec((_TOPK, _BLK), lambda i: (0, i)),
            pl.BlockSpec((_TOPK, _BLK), lambda i: (0, i)),
            pl.BlockSpec((_TOPK, _BLK), lambda i: (0, i)),
        ],
        out_shape=[
            jax.ShapeDtypeStruct((_TOPK, half), jnp.int32),
            jax.ShapeDtypeStruct((_TOPK, half), jnp.float32),
            jax.ShapeDtypeStruct((_TOPK, half), jnp.int32),
            jax.ShapeDtypeStruct((_TOPK, half), jnp.float32),
        ],
    )(x, x, W)
    idx_t = jnp.concatenate([ia, ib], axis=1)
    wgt_t = jnp.concatenate([wa, wb], axis=1)
    return idx_t.T, wgt_t.T
